# baseline (device time: 257107 ns/iter reference)
import jax
import jax.numpy as jnp
from jax import lax
from jax.experimental import pallas as pl
from jax.experimental.pallas import tpu as pltpu

M = 2048
N = 2048
HALF = M // 2


def kernel(dy, W):
    y_idx = lax.axis_index("y")
    dy_half = lax.dynamic_slice_in_dim(dy, y_idx * HALF, HALF, axis=0)
    p = lax.dot_general(
        dy_half.astype(jnp.bfloat16),
        W.astype(jnp.bfloat16),
        (((1,), (1,)), ((), ())),
        preferred_element_type=jnp.float32,
    )
    return _allreduce_xy(p)


def _allreduce_xy(p):
    def body(p_ref, out_ref, xrecv, xsend_sem, xrecv_sem, ysend_sem, yrecv_sem):
        my_x = lax.axis_index("x")
        my_y = lax.axis_index("y")
        my_z = lax.axis_index("z")
        x_nbr = (1 - my_x, my_y, my_z)
        y_nbr = (my_x, 1 - my_y, my_z)

        barrier = pltpu.get_barrier_semaphore()
        for nbr in (x_nbr, y_nbr):
            pl.semaphore_signal(
                barrier, inc=1, device_id=nbr,
                device_id_type=pl.DeviceIdType.MESH,
            )
        pl.semaphore_wait(barrier, 2)

        rx = pltpu.make_async_remote_copy(
            src_ref=p_ref,
            dst_ref=xrecv,
            send_sem=xsend_sem,
            recv_sem=xrecv_sem,
            device_id=x_nbr,
            device_id_type=pl.DeviceIdType.MESH,
        )
        rx.start()
        rx.wait()

        row0 = my_y * HALF
        out_ref[pl.ds(row0, HALF), :] = p_ref[...] + xrecv[...]

        ry = pltpu.make_async_remote_copy(
            src_ref=out_ref.at[pl.ds(row0, HALF), :],
            dst_ref=out_ref.at[pl.ds(row0, HALF), :],
            send_sem=ysend_sem,
            recv_sem=yrecv_sem,
            device_id=y_nbr,
            device_id_type=pl.DeviceIdType.MESH,
        )
        ry.start()
        ry.wait()

    return pl.pallas_call(
        body,
        out_shape=jax.ShapeDtypeStruct((M, N), jnp.float32),
        in_specs=[pl.BlockSpec(memory_space=pltpu.VMEM)],
        out_specs=pl.BlockSpec(memory_space=pltpu.VMEM),
        scratch_shapes=[
            pltpu.VMEM((HALF, N), jnp.float32),
            pltpu.SemaphoreType.DMA,
            pltpu.SemaphoreType.DMA,
            pltpu.SemaphoreType.DMA,
            pltpu.SemaphoreType.DMA,
        ],
        compiler_params=pltpu.CompilerParams(collective_id=0),
    )(p)


# device time: 110976 ns/iter; 2.3168x vs baseline; 2.3168x over previous
import jax
import jax.numpy as jnp
from jax import lax
from jax.experimental import pallas as pl
from jax.experimental.pallas import tpu as pltpu

M = 2048
N = 2048
HALF = M // 2
C = 8
CW = N // C

_MESH = pl.DeviceIdType.MESH


def kernel(dy, W):
    y_idx = lax.axis_index("y")
    dy_half = lax.dynamic_slice_in_dim(dy, y_idx * HALF, HALF, axis=0)
    p = lax.dot_general(
        dy_half.astype(jnp.bfloat16),
        W.astype(jnp.bfloat16),
        (((1,), (1,)), ((), ())),
        preferred_element_type=jnp.float32,
    ).astype(jnp.bfloat16)
    return _allreduce_xy(p)


def _allreduce_xy(p):
    def body(p_ref, out_ref, xrecv, sbuf, yrecv,
             xsend_sems, xrecv_sems, ysend_sems, yrecv_sems,
             xcredit, ycredit):
        my_x = lax.axis_index("x")
        my_y = lax.axis_index("y")
        my_z = lax.axis_index("z")
        x_nbr = (1 - my_x, my_y, my_z)
        y_nbr = (my_x, 1 - my_y, my_z)
        row0 = my_y * HALF
        orow0 = (1 - my_y) * HALF

        barrier = pltpu.get_barrier_semaphore()
        for nbr in (x_nbr, y_nbr):
            pl.semaphore_signal(barrier, inc=1, device_id=nbr,
                                device_id_type=_MESH)
        pl.semaphore_wait(barrier, 2)

        rx, ry = {}, {}

        def x_desc(c):
            return pltpu.make_async_remote_copy(
                src_ref=p_ref.at[:, pl.ds(c * CW, CW)],
                dst_ref=xrecv.at[c % 2],
                send_sem=xsend_sems.at[c % 2], recv_sem=xrecv_sems.at[c % 2],
                device_id=x_nbr, device_id_type=_MESH)

        def y_desc(c):
            return pltpu.make_async_remote_copy(
                src_ref=sbuf.at[c % 2],
                dst_ref=yrecv.at[c % 2],
                send_sem=ysend_sems.at[c % 2], recv_sem=yrecv_sems.at[c % 2],
                device_id=y_nbr, device_id_type=_MESH)

        for i in range(C + 2):
            if i < C:
                c = i
                if c >= 2:
                    rx[c - 2].wait_send()
                    pl.semaphore_wait(xcredit, 1)
                rx[c] = x_desc(c)
                rx[c].start()
            if 1 <= i <= C:
                c = i - 1
                rx[c].wait_recv()
                if c >= 2:
                    ry[c - 2].wait_send()
                s = (p_ref[:, pl.ds(c * CW, CW)].astype(jnp.float32)
                     + xrecv[c % 2].astype(jnp.float32))
                out_ref[pl.ds(row0, HALF), pl.ds(c * CW, CW)] = s
                sbuf[c % 2] = s.astype(jnp.bfloat16)
                pl.semaphore_signal(xcredit, inc=1, device_id=x_nbr,
                                    device_id_type=_MESH)
                if c >= 2:
                    pl.semaphore_wait(ycredit, 1)
                ry[c] = y_desc(c)
                ry[c].start()
            if 2 <= i <= C + 1:
                c = i - 2
                ry[c].wait_recv()
                out_ref[pl.ds(orow0, HALF), pl.ds(c * CW, CW)] = (
                    yrecv[c % 2].astype(jnp.float32))
                pl.semaphore_signal(ycredit, inc=1, device_id=y_nbr,
                                    device_id_type=_MESH)

        rx[C - 2].wait_send()
        rx[C - 1].wait_send()
        ry[C - 2].wait_send()
        ry[C - 1].wait_send()
        pl.semaphore_wait(xcredit, 2)
        pl.semaphore_wait(ycredit, 2)

    return pl.pallas_call(
        body,
        out_shape=jax.ShapeDtypeStruct((M, N), jnp.float32),
        in_specs=[pl.BlockSpec(memory_space=pltpu.VMEM)],
        out_specs=pl.BlockSpec(memory_space=pltpu.VMEM),
        scratch_shapes=[
            pltpu.VMEM((2, HALF, CW), jnp.bfloat16),
            pltpu.VMEM((2, HALF, CW), jnp.bfloat16),
            pltpu.VMEM((2, HALF, CW), jnp.bfloat16),
            pltpu.SemaphoreType.DMA((2,)),
            pltpu.SemaphoreType.DMA((2,)),
            pltpu.SemaphoreType.DMA((2,)),
            pltpu.SemaphoreType.DMA((2,)),
            pltpu.SemaphoreType.REGULAR,
            pltpu.SemaphoreType.REGULAR,
        ],
        compiler_params=pltpu.CompilerParams(collective_id=0),
    )(p)


# device time: 110913 ns/iter; 2.3181x vs baseline; 1.0006x over previous
import jax
import jax.numpy as jnp
from jax import lax
from jax.experimental import pallas as pl
from jax.experimental.pallas import tpu as pltpu

M = 2048
N = 2048
HALF = M // 2
C = 8
CW = N // C

_MESH = pl.DeviceIdType.MESH


def kernel(dy, W):
    y_idx = lax.axis_index("y")
    dy_half = lax.dynamic_slice_in_dim(dy, y_idx * HALF, HALF, axis=0)
    p = lax.dot_general(
        dy_half, W,
        (((1,), (1,)), ((), ())),
        preferred_element_type=jnp.float32,
        precision=lax.Precision.DEFAULT,
    ).astype(jnp.bfloat16)
    return _allreduce_xy(p)


def _allreduce_xy(p):
    def body(p_ref, out_ref, xrecv, sbuf, yrecv,
             xsend_sems, xrecv_sems, ysend_sems, yrecv_sems,
             xcredit, ycredit):
        my_x = lax.axis_index("x")
        my_y = lax.axis_index("y")
        my_z = lax.axis_index("z")
        x_nbr = (1 - my_x, my_y, my_z)
        y_nbr = (my_x, 1 - my_y, my_z)
        row0 = my_y * HALF
        orow0 = (1 - my_y) * HALF

        barrier = pltpu.get_barrier_semaphore()
        for nbr in (x_nbr, y_nbr):
            pl.semaphore_signal(barrier, inc=1, device_id=nbr,
                                device_id_type=_MESH)
        pl.semaphore_wait(barrier, 2)

        rx, ry = {}, {}

        def x_desc(c):
            return pltpu.make_async_remote_copy(
                src_ref=p_ref.at[:, pl.ds(c * CW, CW)],
                dst_ref=xrecv.at[c % 2],
                send_sem=xsend_sems.at[c % 2], recv_sem=xrecv_sems.at[c % 2],
                device_id=x_nbr, device_id_type=_MESH)

        def y_desc(c):
            return pltpu.make_async_remote_copy(
                src_ref=sbuf.at[c % 2],
                dst_ref=yrecv.at[c % 2],
                send_sem=ysend_sems.at[c % 2], recv_sem=yrecv_sems.at[c % 2],
                device_id=y_nbr, device_id_type=_MESH)

        for i in range(C + 2):
            if i < C:
                c = i
                if c >= 2:
                    rx[c - 2].wait_send()
                    pl.semaphore_wait(xcredit, 1)
                rx[c] = x_desc(c)
                rx[c].start()
            if 1 <= i <= C:
                c = i - 1
                rx[c].wait_recv()
                if c >= 2:
                    ry[c - 2].wait_send()
                s = p_ref[:, pl.ds(c * CW, CW)] + xrecv[c % 2]
                sbuf[c % 2] = s
                out_ref[pl.ds(row0, HALF), pl.ds(c * CW, CW)] = (
                    s.astype(jnp.float32))
                pl.semaphore_signal(xcredit, inc=1, device_id=x_nbr,
                                    device_id_type=_MESH)
                if c >= 2:
                    pl.semaphore_wait(ycredit, 1)
                ry[c] = y_desc(c)
                ry[c].start()
            if 2 <= i <= C + 1:
                c = i - 2
                ry[c].wait_recv()
                out_ref[pl.ds(orow0, HALF), pl.ds(c * CW, CW)] = (
                    yrecv[c % 2].astype(jnp.float32))
                pl.semaphore_signal(ycredit, inc=1, device_id=y_nbr,
                                    device_id_type=_MESH)

        rx[C - 2].wait_send()
        rx[C - 1].wait_send()
        ry[C - 2].wait_send()
        ry[C - 1].wait_send()
        pl.semaphore_wait(xcredit, 2)
        pl.semaphore_wait(ycredit, 2)

    return pl.pallas_call(
        body,
        out_shape=jax.ShapeDtypeStruct((M, N), jnp.float32),
        in_specs=[pl.BlockSpec(memory_space=pltpu.VMEM)],
        out_specs=pl.BlockSpec(memory_space=pltpu.VMEM),
        scratch_shapes=[
            pltpu.VMEM((2, HALF, CW), jnp.bfloat16),
            pltpu.VMEM((2, HALF, CW), jnp.bfloat16),
            pltpu.VMEM((2, HALF, CW), jnp.bfloat16),
            pltpu.SemaphoreType.DMA((2,)),
            pltpu.SemaphoreType.DMA((2,)),
            pltpu.SemaphoreType.DMA((2,)),
            pltpu.SemaphoreType.DMA((2,)),
            pltpu.SemaphoreType.REGULAR,
            pltpu.SemaphoreType.REGULAR,
        ],
        compiler_params=pltpu.CompilerParams(collective_id=0),
    )(p)


# device time: 98535 ns/iter; 2.6093x vs baseline; 1.1256x over previous
import jax
import jax.numpy as jnp
from jax import lax
from jax.experimental import pallas as pl
from jax.experimental.pallas import tpu as pltpu

M = 2048
N = 2048
HALF = M // 2
GCOL = N // 2
C = 4
CW = GCOL // C

_MESH = pl.DeviceIdType.MESH


def kernel(dy, W):
    y_idx = lax.axis_index("y")
    zp = lax.axis_index("z") % 2
    dy_half = lax.dynamic_slice_in_dim(dy, y_idx * HALF, HALF, axis=0)
    w_own = lax.dynamic_slice_in_dim(W, zp * GCOL, GCOL, axis=0)
    p = lax.dot_general(
        dy_half, w_own,
        (((1,), (1,)), ((), ())),
        preferred_element_type=jnp.float32,
        precision=lax.Precision.DEFAULT,
    ).astype(jnp.bfloat16)
    return _allreduce_xyz(p)


def _allreduce_xyz(p):
    def body(p_ref, out_ref, xrecv, yrecv, zsend, zrecv,
             xsend_sems, xrecv_sems, ysend_sems, yrecv_sems,
             zsend_sems, zrecv_sems):
        my_x = lax.axis_index("x")
        my_y = lax.axis_index("y")
        my_z = lax.axis_index("z")
        zp = my_z % 2
        x_nbr = (1 - my_x, my_y, my_z)
        y_nbr = (my_x, 1 - my_y, my_z)
        z_nbr = (my_x, my_y, my_z + 1 - 2 * zp)
        row0 = my_y * HALF
        orow0 = (1 - my_y) * HALF
        g0 = zp * GCOL
        og0 = (1 - zp) * GCOL

        barrier = pltpu.get_barrier_semaphore()
        for nbr in (x_nbr, y_nbr, z_nbr):
            pl.semaphore_signal(barrier, inc=1, device_id=nbr,
                                device_id_type=_MESH)
        pl.semaphore_wait(barrier, 3)

        rx, ry, rz = {}, {}, {}

        def x_desc(c):
            return pltpu.make_async_remote_copy(
                src_ref=p_ref.at[:, pl.ds(c * CW, CW)],
                dst_ref=xrecv.at[c],
                send_sem=xsend_sems.at[c], recv_sem=xrecv_sems.at[c],
                device_id=x_nbr, device_id_type=_MESH)

        def y_desc(c):
            return pltpu.make_async_remote_copy(
                src_ref=zsend.at[c, pl.ds(row0, HALF), :],
                dst_ref=yrecv.at[c],
                send_sem=ysend_sems.at[c], recv_sem=yrecv_sems.at[c],
                device_id=y_nbr, device_id_type=_MESH)

        def z_desc(c):
            return pltpu.make_async_remote_copy(
                src_ref=zsend.at[c],
                dst_ref=zrecv.at[c],
                send_sem=zsend_sems.at[c], recv_sem=zrecv_sems.at[c],
                device_id=z_nbr, device_id_type=_MESH)

        for i in range(C + 3):
            if i < C:
                c = i
                rx[c] = x_desc(c)
                rx[c].start()
            if 1 <= i <= C:
                c = i - 1
                rx[c].wait_recv()
                s = p_ref[:, pl.ds(c * CW, CW)] + xrecv[c]
                zsend[c, pl.ds(row0, HALF), :] = s
                out_ref[pl.ds(row0, HALF), pl.ds(g0 + c * CW, CW)] = (
                    s.astype(jnp.float32))
                ry[c] = y_desc(c)
                ry[c].start()
            if 2 <= i <= C + 1:
                c = i - 2
                ry[c].wait_recv()
                yv = yrecv[c]
                zsend[c, pl.ds(orow0, HALF), :] = yv
                out_ref[pl.ds(orow0, HALF), pl.ds(g0 + c * CW, CW)] = (
                    yv.astype(jnp.float32))
                rz[c] = z_desc(c)
                rz[c].start()
            if 3 <= i <= C + 2:
                c = i - 3
                rz[c].wait_recv()
                out_ref[:, pl.ds(og0 + c * CW, CW)] = (
                    zrecv[c].astype(jnp.float32))

        for c in range(C):
            rx[c].wait_send()
            ry[c].wait_send()
            rz[c].wait_send()

    return pl.pallas_call(
        body,
        out_shape=jax.ShapeDtypeStruct((M, N), jnp.float32),
        in_specs=[pl.BlockSpec(memory_space=pltpu.VMEM)],
        out_specs=pl.BlockSpec(memory_space=pltpu.VMEM),
        scratch_shapes=[
            pltpu.VMEM((C, HALF, CW), jnp.bfloat16),
            pltpu.VMEM((C, HALF, CW), jnp.bfloat16),
            pltpu.VMEM((C, M, CW), jnp.bfloat16),
            pltpu.VMEM((C, M, CW), jnp.bfloat16),
            pltpu.SemaphoreType.DMA((C,)),
            pltpu.SemaphoreType.DMA((C,)),
            pltpu.SemaphoreType.DMA((C,)),
            pltpu.SemaphoreType.DMA((C,)),
            pltpu.SemaphoreType.DMA((C,)),
            pltpu.SemaphoreType.DMA((C,)),
        ],
        compiler_params=pltpu.CompilerParams(collective_id=0),
    )(p)


# device time: 91417 ns/iter; 2.8125x vs baseline; 1.0779x over previous
import jax
import jax.numpy as jnp
from jax import lax
from jax.experimental import pallas as pl
from jax.experimental.pallas import tpu as pltpu

M = 2048
N = 2048
HALF = M // 2
GCOL = N // 2
C = 4
CW = GCOL // C

_MESH = pl.DeviceIdType.MESH


def kernel(dy, W):
    y_idx = lax.axis_index("y")
    zp = lax.axis_index("z") % 2
    dy_half = lax.dynamic_slice_in_dim(dy, y_idx * HALF, HALF, axis=0)
    w_own = lax.dynamic_slice_in_dim(W, zp * GCOL, GCOL, axis=0)
    p = lax.dot_general(
        dy_half, w_own,
        (((1,), (1,)), ((), ())),
        preferred_element_type=jnp.float32,
        precision=lax.Precision.DEFAULT,
    ).astype(jnp.bfloat16)
    return _allreduce_xyz(p)


def _allreduce_xyz(p):
    def body(p_ref, out_ref, xrecv, yrecv, zsend, zrecv,
             xsend_sems, xrecv_sems, ysend_sems, yrecv_sems,
             zsend_sems, zrecv_sems):
        my_x = lax.axis_index("x")
        my_y = lax.axis_index("y")
        my_z = lax.axis_index("z")
        zp = my_z % 2
        x_nbr = (1 - my_x, my_y, my_z)
        y_nbr = (my_x, 1 - my_y, my_z)
        z_nbr = (my_x, my_y, my_z + 1 - 2 * zp)
        row0 = my_y * HALF
        orow0 = (1 - my_y) * HALF
        g0 = zp * GCOL
        og0 = (1 - zp) * GCOL

        barrier = pltpu.get_barrier_semaphore()
        for nbr in (x_nbr, y_nbr, z_nbr):
            pl.semaphore_signal(barrier, inc=1, device_id=nbr,
                                device_id_type=_MESH)
        pl.semaphore_wait(barrier, 3)

        rx, ry, rza, rzb = {}, {}, {}, {}

        def x_desc(c):
            return pltpu.make_async_remote_copy(
                src_ref=p_ref.at[:, pl.ds(c * CW, CW)],
                dst_ref=xrecv.at[c],
                send_sem=xsend_sems.at[c], recv_sem=xrecv_sems.at[c],
                device_id=x_nbr, device_id_type=_MESH)

        def y_desc(c):
            return pltpu.make_async_remote_copy(
                src_ref=zsend.at[c, pl.ds(row0, HALF), :],
                dst_ref=yrecv.at[c],
                send_sem=ysend_sems.at[c], recv_sem=yrecv_sems.at[c],
                device_id=y_nbr, device_id_type=_MESH)

        def z_desc(c, r0):
            return pltpu.make_async_remote_copy(
                src_ref=zsend.at[c, pl.ds(r0, HALF), :],
                dst_ref=zrecv.at[c, pl.ds(r0, HALF), :],
                send_sem=zsend_sems.at[c], recv_sem=zrecv_sems.at[c],
                device_id=z_nbr, device_id_type=_MESH)

        for i in range(C + 3):
            if i < C:
                c = i
                rx[c] = x_desc(c)
                rx[c].start()
            if 1 <= i <= C:
                c = i - 1
                rx[c].wait_recv()
                s = p_ref[:, pl.ds(c * CW, CW)] + xrecv[c]
                zsend[c, pl.ds(row0, HALF), :] = s
                out_ref[pl.ds(row0, HALF), pl.ds(g0 + c * CW, CW)] = (
                    s.astype(jnp.float32))
                ry[c] = y_desc(c)
                ry[c].start()
                rza[c] = z_desc(c, row0)
                rza[c].start()
            if 2 <= i <= C + 1:
                c = i - 2
                ry[c].wait_recv()
                yv = yrecv[c]
                zsend[c, pl.ds(orow0, HALF), :] = yv
                out_ref[pl.ds(orow0, HALF), pl.ds(g0 + c * CW, CW)] = (
                    yv.astype(jnp.float32))
                rzb[c] = z_desc(c, orow0)
                rzb[c].start()
            if 3 <= i <= C + 2:
                c = i - 3
                rza[c].wait_recv()
                rzb[c].wait_recv()
                out_ref[:, pl.ds(og0 + c * CW, CW)] = (
                    zrecv[c].astype(jnp.float32))

        for c in range(C):
            rx[c].wait_send()
            ry[c].wait_send()
            rza[c].wait_send()
            rzb[c].wait_send()

    return pl.pallas_call(
        body,
        out_shape=jax.ShapeDtypeStruct((M, N), jnp.float32),
        in_specs=[pl.BlockSpec(memory_space=pltpu.VMEM)],
        out_specs=pl.BlockSpec(memory_space=pltpu.VMEM),
        scratch_shapes=[
            pltpu.VMEM((C, HALF, CW), jnp.bfloat16),
            pltpu.VMEM((C, HALF, CW), jnp.bfloat16),
            pltpu.VMEM((C, M, CW), jnp.bfloat16),
            pltpu.VMEM((C, M, CW), jnp.bfloat16),
            pltpu.SemaphoreType.DMA((C,)),
            pltpu.SemaphoreType.DMA((C,)),
            pltpu.SemaphoreType.DMA((C,)),
            pltpu.SemaphoreType.DMA((C,)),
            pltpu.SemaphoreType.DMA((C,)),
            pltpu.SemaphoreType.DMA((C,)),
        ],
        compiler_params=pltpu.CompilerParams(collective_id=0),
    )(p)
